# baseline (speedup 1.0000x reference)
import functools

import jax
import jax.numpy as jnp
from jax import lax
from jax.experimental import pallas as pl
from jax.experimental.pallas import tpu as pltpu
from jax.experimental.pallas import tpu_sc as plsc

VOCAB = 100000
D_MODEL = 4096
BATCH = 4
SEQ = 2048
N_TOK = BATCH * SEQ

NUM_WORKERS = 32
TOK_PER_W = N_TOK // NUM_WORKERS
CHUNK = 8
NCHUNKS = TOK_PER_W // CHUNK
HALVES = 2
DH = D_MODEL // HALVES
NBUF = 7
PRE = 6

MASK_MIN = float(jnp.finfo(jnp.float32).min)
MASK_BLK = 256


def _gather_sc(ids, table):
    mesh = plsc.VectorSubcoreMesh(core_axis_name="c", subcore_axis_name="s")
    assert PRE < NBUF and TOK_PER_W % CHUNK == 0
    w_per_batch = SEQ // TOK_PER_W

    @functools.partial(
        pl.kernel,
        mesh=mesh,
        out_type=jax.ShapeDtypeStruct((N_TOK, D_MODEL), jnp.float32),
        scratch_types=(
            [pltpu.VMEM((TOK_PER_W,), jnp.int32)]
            + [pltpu.VMEM((CHUNK, DH), jnp.float32)] * NBUF
            + [pltpu.SemaphoreType.DMA] * (2 * NBUF)
        ),
    )
    def k(ids_hbm, table_hbm, out_hbm, idx_v, *bufs_and_sems):
        rows = bufs_and_sems[:NBUF]
        gsems = bufs_and_sems[NBUF : 2 * NBUF]
        osems = bufs_and_sems[2 * NBUF :]
        wid = lax.axis_index("s") * 2 + lax.axis_index("c")
        base = pl.multiple_of(wid * TOK_PER_W, TOK_PER_W)
        b = wid // w_per_batch
        off = pl.multiple_of((wid % w_per_batch) * TOK_PER_W, TOK_PER_W)
        pltpu.sync_copy(ids_hbm.at[b, pl.ds(off, TOK_PER_W)], idx_v)

        NQ = NCHUNKS * HALVES

        def src(q):
            tc, h = q // HALVES, q % HALVES
            return table_hbm.at[
                idx_v.at[pl.ds(tc * CHUNK, CHUNK)], pl.ds(h * DH, DH)
            ]

        def dst(q):
            tc, h = q // HALVES, q % HALVES
            return out_hbm.at[pl.ds(base + tc * CHUNK, CHUNK), pl.ds(h * DH, DH)]

        def fire_gather(q):
            pltpu.async_copy(src(q), rows[q % NBUF], gsems[q % NBUF])

        def wait_gather(q):
            pltpu.make_async_copy(src(q), rows[q % NBUF], gsems[q % NBUF]).wait()

        def fire_out(q):
            pltpu.async_copy(rows[q % NBUF], dst(q), osems[q % NBUF])

        def wait_out(q):
            pltpu.make_async_copy(rows[q % NBUF], dst(q), osems[q % NBUF]).wait()

        for p in range(PRE):
            fire_gather(p)
        for q in range(NQ):
            wait_gather(q)
            fire_out(q)
            nq = q + PRE
            if nq < NQ:
                if nq - NBUF >= 0:
                    wait_out(nq - NBUF)
                fire_gather(nq)
        for q in range(max(0, NQ - NBUF), NQ):
            wait_out(q)

    return k(ids, table)


def _mask_body(mask_ref, pos_ref):
    j = pl.program_id(1)
    row = lax.broadcasted_iota(jnp.int32, (MASK_BLK, SEQ), 0) + j * MASK_BLK
    col = lax.broadcasted_iota(jnp.int32, (MASK_BLK, SEQ), 1)
    mask_ref[0, 0] = jnp.where(col <= row, 0.0, MASK_MIN).astype(jnp.float32)
    pos_ref[...] = lax.broadcasted_iota(jnp.int32, (1, SEQ), 1)


def _mask_tc():
    return pl.pallas_call(
        _mask_body,
        grid=(BATCH, SEQ // MASK_BLK),
        out_shape=(
            jax.ShapeDtypeStruct((BATCH, 1, SEQ, SEQ), jnp.float32),
            jax.ShapeDtypeStruct((1, SEQ), jnp.int32),
        ),
        out_specs=(
            pl.BlockSpec((1, 1, MASK_BLK, SEQ), lambda b, j: (b, 0, j, 0)),
            pl.BlockSpec((1, SEQ), lambda b, j: (0, 0)),
        ),
    )()


def kernel(input_ids, embed_tokens):
    embeds = _gather_sc(input_ids.astype(jnp.int32), embed_tokens)
    hidden = embeds.reshape(BATCH, SEQ, D_MODEL)
    combined_mask, position_ids = _mask_tc()
    return (hidden, combined_mask, position_ids)


# speedup vs baseline: 1.0053x; 1.0053x over previous
import functools

import jax
import jax.numpy as jnp
from jax import lax
from jax.experimental import pallas as pl
from jax.experimental.pallas import tpu as pltpu
from jax.experimental.pallas import tpu_sc as plsc

VOCAB = 100000
D_MODEL = 4096
BATCH = 4
SEQ = 2048
N_TOK = BATCH * SEQ

NUM_WORKERS = 32
TOK_PER_W = N_TOK // NUM_WORKERS
CHUNK = 8
NCHUNKS = TOK_PER_W // CHUNK
HALVES = 4
DH = D_MODEL // HALVES
NBUF = 15
PRE = 12

MASK_MIN = float(jnp.finfo(jnp.float32).min)
MASK_BLK = 256


def _gather_sc(ids, table):
    mesh = plsc.VectorSubcoreMesh(core_axis_name="c", subcore_axis_name="s")
    assert PRE < NBUF and TOK_PER_W % CHUNK == 0
    w_per_batch = SEQ // TOK_PER_W

    @functools.partial(
        pl.kernel,
        mesh=mesh,
        out_type=jax.ShapeDtypeStruct((N_TOK, D_MODEL), jnp.float32),
        scratch_types=(
            [pltpu.VMEM((TOK_PER_W,), jnp.int32)]
            + [pltpu.VMEM((CHUNK, DH), jnp.float32)] * NBUF
            + [pltpu.SemaphoreType.DMA] * (2 * NBUF)
        ),
    )
    def k(ids_hbm, table_hbm, out_hbm, idx_v, *bufs_and_sems):
        rows = bufs_and_sems[:NBUF]
        gsems = bufs_and_sems[NBUF : 2 * NBUF]
        osems = bufs_and_sems[2 * NBUF :]
        wid = lax.axis_index("s") * 2 + lax.axis_index("c")
        base = pl.multiple_of(wid * TOK_PER_W, TOK_PER_W)
        b = wid // w_per_batch
        off = pl.multiple_of((wid % w_per_batch) * TOK_PER_W, TOK_PER_W)
        pltpu.sync_copy(ids_hbm.at[b, pl.ds(off, TOK_PER_W)], idx_v)

        NQ = NCHUNKS * HALVES

        def src(q):
            tc, h = q // HALVES, q % HALVES
            return table_hbm.at[
                idx_v.at[pl.ds(tc * CHUNK, CHUNK)], pl.ds(h * DH, DH)
            ]

        def dst(q):
            tc, h = q // HALVES, q % HALVES
            return out_hbm.at[pl.ds(base + tc * CHUNK, CHUNK), pl.ds(h * DH, DH)]

        def fire_gather(q):
            pltpu.async_copy(src(q), rows[q % NBUF], gsems[q % NBUF])

        def wait_gather(q):
            pltpu.make_async_copy(src(q), rows[q % NBUF], gsems[q % NBUF]).wait()

        def fire_out(q):
            pltpu.async_copy(rows[q % NBUF], dst(q), osems[q % NBUF])

        def wait_out(q):
            pltpu.make_async_copy(rows[q % NBUF], dst(q), osems[q % NBUF]).wait()

        for p in range(PRE):
            fire_gather(p)
        for q in range(NQ):
            wait_gather(q)
            fire_out(q)
            nq = q + PRE
            if nq < NQ:
                if nq - NBUF >= 0:
                    wait_out(nq - NBUF)
                fire_gather(nq)
        for q in range(max(0, NQ - NBUF), NQ):
            wait_out(q)

    return k(ids, table)


def _mask_body(mask_ref, pos_ref):
    j = pl.program_id(1)
    row = lax.broadcasted_iota(jnp.int32, (MASK_BLK, SEQ), 0) + j * MASK_BLK
    col = lax.broadcasted_iota(jnp.int32, (MASK_BLK, SEQ), 1)
    mask_ref[0, 0] = jnp.where(col <= row, 0.0, MASK_MIN).astype(jnp.float32)
    pos_ref[...] = lax.broadcasted_iota(jnp.int32, (1, SEQ), 1)


def _mask_tc():
    return pl.pallas_call(
        _mask_body,
        grid=(BATCH, SEQ // MASK_BLK),
        out_shape=(
            jax.ShapeDtypeStruct((BATCH, 1, SEQ, SEQ), jnp.float32),
            jax.ShapeDtypeStruct((1, SEQ), jnp.int32),
        ),
        out_specs=(
            pl.BlockSpec((1, 1, MASK_BLK, SEQ), lambda b, j: (b, 0, j, 0)),
            pl.BlockSpec((1, SEQ), lambda b, j: (0, 0)),
        ),
    )()


def kernel(input_ids, embed_tokens):
    embeds = _gather_sc(input_ids.astype(jnp.int32), embed_tokens)
    hidden = embeds.reshape(BATCH, SEQ, D_MODEL)
    combined_mask, position_ids = _mask_tc()
    return (hidden, combined_mask, position_ids)
